# baseline (speedup 1.0000x reference)
import functools

import jax
import jax.numpy as jnp
from jax import lax
from jax.experimental import pallas as pl
from jax.experimental.pallas import tpu as pltpu
from jax.experimental.pallas import tpu_sc as plsc

_NC = 2
_NS = 16
_NW = _NC * _NS
_CH = 80
_NB = 4


@functools.lru_cache(maxsize=None)
def _make_segsum(n, e, d):
    ept = e // _NW
    nchunk = ept // _CH
    assert ept * _NW == e and nchunk * _CH == ept
    assert nchunk >= 2 * _NB and (nchunk - 2 - 3) % _NB == 0
    npad = -(-n // (_NS * 8)) * (_NS * 8)
    rpt = npad // _NS
    mesh = plsc.VectorSubcoreMesh(core_axis_name="c", subcore_axis_name="s")

    @functools.partial(
        pl.kernel,
        mesh=mesh,
        out_type=jax.ShapeDtypeStruct((_NC, npad, d), jnp.float32),
        scratch_types=[
            [pltpu.VMEM((_CH,), jnp.int32) for _ in range(_NB)],
            [pltpu.VMEM((_CH,), jnp.int32) for _ in range(_NB)],
            [pltpu.VMEM((_CH, d), jnp.float32) for _ in range(_NB)],
            pltpu.VMEM_SHARED((npad, d), jnp.float32),
            [pltpu.SemaphoreType.DMA for _ in range(_NB)],
            [pltpu.SemaphoreType.DMA for _ in range(_NB)],
            [pltpu.SemaphoreType.DMA for _ in range(_NB)],
            [pltpu.SemaphoreType.DMA for _ in range(_NB)],
        ],
    )
    def segsum(x_hbm, src_hbm, dst_hbm, zeros_hbm, out_hbm,
               sbuf, dbuf, rows, agg_sh, asem, bsem, gsem, ssem):
        c = lax.axis_index("c")
        s = lax.axis_index("s")
        base = (s * _NC + c) * ept
        r0 = s * rpt
        pltpu.sync_copy(zeros_hbm.at[pl.ds(r0, rpt)], agg_sh.at[pl.ds(r0, rpt)])
        plsc.subcore_barrier()

        def src_cp(j, k):
            return pltpu.make_async_copy(
                src_hbm.at[pl.ds(base + j * _CH, _CH)], sbuf[k], asem[k])

        def dst_cp(j, k):
            return pltpu.make_async_copy(
                dst_hbm.at[pl.ds(base + j * _CH, _CH)], dbuf[k], bsem[k])

        def g_cp(k):
            return pltpu.make_async_copy(x_hbm.at[sbuf[k]], rows[k], gsem[k])

        def s_sync(k):
            pltpu.async_copy(rows[k], agg_sh.at[dbuf[k]], ssem[k],
                             add=True).wait()

        def step(j, k, a_st=True, b_st=True, g_st=True):
            kn1 = (k + 1) % _NB
            kn2 = (k + 2) % _NB
            kn3 = (k + 3) % _NB
            g_cp(k).wait()
            dst_cp(j, k).wait()
            s_sync(k)
            if a_st:
                src_cp(j + 3, kn3).start()
            if b_st:
                dst_cp(j + 1, kn1).start()
            if g_st:
                src_cp(j + 2, kn2).wait()
                g_cp(kn2).start()

        src_cp(0, 0).start()
        src_cp(1, 1).start()
        src_cp(2, 2).start()
        src_cp(0, 0).wait()
        g_cp(0).start()
        src_cp(1, 1).wait()
        g_cp(1).start()
        dst_cp(0, 0).start()

        step(0, 0)
        step(1, 1)

        def body(i, carry):
            j = 2 + i * _NB
            for m in range(_NB):
                step(j + m, (2 + m) % _NB)
            return carry

        lax.fori_loop(0, (nchunk - 5) // _NB, body, 0)
        j = nchunk - 3
        step(j, j % _NB, a_st=False)
        step(j + 1, (j + 1) % _NB, a_st=False, g_st=False)
        step(j + 2, (j + 2) % _NB, a_st=False, b_st=False, g_st=False)

        plsc.subcore_barrier()
        pltpu.sync_copy(agg_sh.at[pl.ds(r0, rpt)],
                        out_hbm.at[c, pl.ds(r0, rpt)])

    return segsum


@functools.lru_cache(maxsize=None)
def _make_dense(n, d, h, bn, res):
    def body(*refs):
        it = iter(refs)
        x_ref, agg_ref, w1, b1, w2, b2 = (next(it) for _ in range(6))
        g, b = (next(it), next(it)) if bn else (None, None)
        r_ref = next(it) if res else None
        o_ref = next(it)
        hs = x_ref[...] + agg_ref[0, :n, :] + agg_ref[1, :n, :]
        t = jnp.maximum(
            jnp.dot(hs, w1[...], preferred_element_type=jnp.float32) + b1[...], 0.0)
        y = jnp.dot(t, w2[...], preferred_element_type=jnp.float32) + b2[...]
        if bn:
            m = jnp.mean(y, axis=0, keepdims=True)
            v = jnp.mean((y - m) ** 2, axis=0, keepdims=True)
            y = (y - m) * lax.rsqrt(v + 1e-5) * g[...] + b[...]
            y = jnp.maximum(y, 0.0)
        if res:
            y = y + r_ref[...]
        o_ref[...] = y

    return pl.pallas_call(
        body,
        out_shape=jax.ShapeDtypeStruct((n, d), jnp.float32),
    )


def _conv(x, agg, w1, b1, w2, b2, gamma=None, beta=None, res=None):
    n, d = x.shape
    h = w1.shape[1]
    bn = gamma is not None
    args = [x, agg, w1, b1.reshape(1, h), w2, b2.reshape(1, -1)]
    if bn:
        args += [gamma.reshape(1, -1), beta.reshape(1, -1)]
    if res is not None:
        args += [res]
    return _make_dense(n, d, h, bn, res is not None)(*args)


def kernel(x, edge_index, params):
    n, d = x.shape
    e = edge_index.shape[1]
    src = edge_index[0]
    dst = edge_index[1]
    npad = -(-n // (_NS * 8)) * (_NS * 8)
    zeros = jnp.zeros((npad, d), jnp.float32)
    segsum = _make_segsum(n, e, d)

    def agg_of(xin):
        return segsum(xin, src, dst, zeros)

    p = params["head"]
    out = _conv(x, agg_of(x), p["conv"]["W1"], p["conv"]["b1"],
                p["conv"]["W2"], p["conv"]["b2"], p["gamma"], p["beta"])
    for name in ("res1", "res2", "res3"):
        p = params[name]
        o1 = _conv(out, agg_of(out), p["conv1"]["W1"], p["conv1"]["b1"],
                   p["conv1"]["W2"], p["conv1"]["b2"], p["gamma1"], p["beta1"])
        out = _conv(o1, agg_of(o1), p["conv2"]["W1"], p["conv2"]["b1"],
                    p["conv2"]["W2"], p["conv2"]["b2"], p["gamma2"], p["beta2"],
                    res=out)
    p = params["tail"]["conv"]
    hid = p["W1"].shape[1]
    dout = p["W2"].shape[1]
    w1p = jnp.pad(p["W1"], ((0, 0), (0, d - hid)))
    b1p = jnp.pad(p["b1"], (0, d - hid))
    w2p = jnp.pad(p["W2"], ((0, d - hid), (0, d - dout)))
    b2p = jnp.pad(p["b2"], (0, d - dout))
    y = _conv(out, agg_of(out), w1p, b1p, w2p, b2p)
    return y[:, :dout]


# speedup vs baseline: 1.2460x; 1.2460x over previous
import functools

import jax
import jax.numpy as jnp
from jax import lax
from jax.experimental import pallas as pl
from jax.experimental.pallas import tpu as pltpu
from jax.experimental.pallas import tpu_sc as plsc

_NC = 2
_NS = 16
_NW = _NC * _NS
_CH = 80
_NB = 3


@functools.lru_cache(maxsize=None)
def _make_segsum(n, e, d):
    ept = e // _NW
    nchunk = ept // _CH
    assert ept * _NW == e and nchunk * _CH == ept
    assert nchunk >= 2 * _NB and (nchunk - 2 - 3) % _NB == 0
    npad = -(-n // (_NS * 8)) * (_NS * 8)
    rpt = npad // _NS
    mesh = plsc.VectorSubcoreMesh(core_axis_name="c", subcore_axis_name="s")

    @functools.partial(
        pl.kernel,
        mesh=mesh,
        out_type=jax.ShapeDtypeStruct((_NC, npad, d), jnp.float32),
        scratch_types=[
            [pltpu.VMEM((_CH,), jnp.int32) for _ in range(_NB)],
            pltpu.VMEM((nchunk, _CH), jnp.int32),
            [pltpu.VMEM((_CH, d), jnp.float32) for _ in range(_NB)],
            pltpu.VMEM_SHARED((npad, d), jnp.float32),
            [pltpu.SemaphoreType.DMA for _ in range(_NB)],
            [pltpu.SemaphoreType.DMA for _ in range(_NB)],
            [pltpu.SemaphoreType.DMA for _ in range(_NB)],
        ],
    )
    def segsum(x_hbm, src_hbm, dst_hbm, zeros_hbm, out_hbm,
               sbuf, dst_v, rows, agg_sh, asem, gsem, ssem):
        c = lax.axis_index("c")
        s = lax.axis_index("s")
        wid = s * _NC + c
        base = wid * ept
        pltpu.sync_copy(dst_hbm.at[wid], dst_v)
        r0 = s * rpt
        pltpu.sync_copy(zeros_hbm.at[pl.ds(r0, rpt)], agg_sh.at[pl.ds(r0, rpt)])
        plsc.subcore_barrier()

        def src_cp(j, k):
            return pltpu.make_async_copy(
                src_hbm.at[pl.ds(base + j * _CH, _CH)], sbuf[k], asem[k])

        def g_cp(k):
            return pltpu.make_async_copy(x_hbm.at[sbuf[k]], rows[k], gsem[k])

        def s_start(j, k):
            pltpu.async_copy(rows[k], agg_sh.at[dst_v.at[j]], ssem[k],
                             add=True)

        def s_drain(j, k):
            pltpu.make_async_copy(rows[k], agg_sh.at[dst_v.at[j]],
                                  ssem[k]).wait()

        def step(j, k, s_dr=True, a_st=True, g_st=True):
            kp = (k + _NB - 1) % _NB
            kn2 = (k + 2) % _NB
            if s_dr:
                s_drain(j - 1, kp)
            g_cp(k).wait()
            s_start(j, k)
            if a_st:
                src_cp(j + 3, k).start()
            if g_st:
                src_cp(j + 2, kn2).wait()
                g_cp(kn2).start()

        src_cp(0, 0).start()
        src_cp(1, 1).start()
        src_cp(2, 2).start()
        src_cp(0, 0).wait()
        g_cp(0).start()
        src_cp(1, 1).wait()
        g_cp(1).start()

        step(0, 0, s_dr=False)
        step(1, 1)

        def body(i, carry):
            j0 = 2 + i * _NB
            for m in range(_NB):
                step(j0 + m, (2 + m) % _NB)
            return carry

        lax.fori_loop(0, (nchunk - 5) // _NB, body, 0)
        j = nchunk - 3
        step(j, j % _NB, a_st=False)
        step(j + 1, (j + 1) % _NB, a_st=False, g_st=False)
        step(j + 2, (j + 2) % _NB, a_st=False, g_st=False)
        s_drain(nchunk - 1, (nchunk - 1) % _NB)

        plsc.subcore_barrier()
        pltpu.sync_copy(agg_sh.at[pl.ds(r0, rpt)],
                        out_hbm.at[c, pl.ds(r0, rpt)])

    return segsum


@functools.lru_cache(maxsize=None)
def _make_dense(n, d, h, bn, res):
    def body(*refs):
        it = iter(refs)
        x_ref, agg_ref, w1, b1, w2, b2 = (next(it) for _ in range(6))
        g, b = (next(it), next(it)) if bn else (None, None)
        r_ref = next(it) if res else None
        o_ref = next(it)
        hs = x_ref[...] + agg_ref[0, :n, :] + agg_ref[1, :n, :]
        t = jnp.maximum(
            jnp.dot(hs, w1[...], preferred_element_type=jnp.float32) + b1[...], 0.0)
        y = jnp.dot(t, w2[...], preferred_element_type=jnp.float32) + b2[...]
        if bn:
            m = jnp.mean(y, axis=0, keepdims=True)
            v = jnp.mean((y - m) ** 2, axis=0, keepdims=True)
            y = (y - m) * lax.rsqrt(v + 1e-5) * g[...] + b[...]
            y = jnp.maximum(y, 0.0)
        if res:
            y = y + r_ref[...]
        o_ref[...] = y

    return pl.pallas_call(
        body,
        out_shape=jax.ShapeDtypeStruct((n, d), jnp.float32),
    )


def _conv(x, agg, w1, b1, w2, b2, gamma=None, beta=None, res=None):
    n, d = x.shape
    h = w1.shape[1]
    bn = gamma is not None
    args = [x, agg, w1, b1.reshape(1, h), w2, b2.reshape(1, -1)]
    if bn:
        args += [gamma.reshape(1, -1), beta.reshape(1, -1)]
    if res is not None:
        args += [res]
    return _make_dense(n, d, h, bn, res is not None)(*args)


def kernel(x, edge_index, params):
    n, d = x.shape
    e = edge_index.shape[1]
    src = edge_index[0]
    dst = edge_index[1]
    npad = -(-n // (_NS * 8)) * (_NS * 8)
    zeros = jnp.zeros((npad, d), jnp.float32)
    dst3 = dst.reshape(_NW, e // (_NW * _CH), _CH)
    segsum = _make_segsum(n, e, d)

    def agg_of(xin):
        return segsum(xin, src, dst3, zeros)

    p = params["head"]
    out = _conv(x, agg_of(x), p["conv"]["W1"], p["conv"]["b1"],
                p["conv"]["W2"], p["conv"]["b2"], p["gamma"], p["beta"])
    for name in ("res1", "res2", "res3"):
        p = params[name]
        o1 = _conv(out, agg_of(out), p["conv1"]["W1"], p["conv1"]["b1"],
                   p["conv1"]["W2"], p["conv1"]["b2"], p["gamma1"], p["beta1"])
        out = _conv(o1, agg_of(o1), p["conv2"]["W1"], p["conv2"]["b1"],
                    p["conv2"]["W2"], p["conv2"]["b2"], p["gamma2"], p["beta2"],
                    res=out)
    p = params["tail"]["conv"]
    hid = p["W1"].shape[1]
    dout = p["W2"].shape[1]
    w1p = jnp.pad(p["W1"], ((0, 0), (0, d - hid)))
    b1p = jnp.pad(p["b1"], (0, d - hid))
    w2p = jnp.pad(p["W2"], ((0, d - hid), (0, d - dout)))
    b2p = jnp.pad(p["b2"], (0, d - dout))
    y = _conv(out, agg_of(out), w1p, b1p, w2p, b2p)
    return y[:, :dout]


# speedup vs baseline: 1.2745x; 1.0229x over previous
import functools

import jax
import jax.numpy as jnp
from jax import lax
from jax.experimental import pallas as pl
from jax.experimental.pallas import tpu as pltpu
from jax.experimental.pallas import tpu_sc as plsc

_NC = 2
_NS = 16
_NW = _NC * _NS
_CH = 80
_NB = 4


@functools.lru_cache(maxsize=None)
def _make_segsum(n, e, d):
    ept = e // _NW
    nchunk = ept // _CH
    assert ept * _NW == e and nchunk * _CH == ept
    assert nchunk >= 2 * _NB and (nchunk - 2 - 3) % _NB == 0
    npad = -(-n // (_NS * 8)) * (_NS * 8)
    rpt = npad // _NS
    mesh = plsc.VectorSubcoreMesh(core_axis_name="c", subcore_axis_name="s")

    @functools.partial(
        pl.kernel,
        mesh=mesh,
        out_type=jax.ShapeDtypeStruct((_NC, npad, d), jnp.float32),
        scratch_types=[
            [pltpu.VMEM((_CH,), jnp.int32) for _ in range(_NB)],
            [pltpu.VMEM((_CH,), jnp.int32) for _ in range(_NB)],
            [pltpu.VMEM((_CH, d), jnp.float32) for _ in range(_NB)],
            pltpu.VMEM_SHARED((npad, d), jnp.float32),
            [pltpu.SemaphoreType.DMA for _ in range(_NB)],
            [pltpu.SemaphoreType.DMA for _ in range(_NB)],
            [pltpu.SemaphoreType.DMA for _ in range(_NB)],
            [pltpu.SemaphoreType.DMA for _ in range(_NB)],
        ],
    )
    def segsum(x_hbm, src_hbm, dst_hbm, zeros_hbm, out_hbm,
               sbuf, dbuf, rows, agg_sh, asem, bsem, gsem, ssem):
        c = lax.axis_index("c")
        s = lax.axis_index("s")
        base = (s * _NC + c) * ept
        r0 = s * rpt

        def src_cp(j, k):
            return pltpu.make_async_copy(
                src_hbm.at[pl.ds(base + j * _CH, _CH)], sbuf[k], asem[k])

        def dst_cp(j, k):
            return pltpu.make_async_copy(
                dst_hbm.at[pl.ds(base + j * _CH, _CH)], dbuf[k], bsem[k])

        def g_cp(k):
            return pltpu.make_async_copy(x_hbm.at[sbuf[k]], rows[k], gsem[k])

        def s_start(k):
            pltpu.async_copy(rows[k], agg_sh.at[dbuf[k]], ssem[k], add=True)

        def s_drain(k):
            pltpu.make_async_copy(rows[k], agg_sh.at[dbuf[k]], ssem[k]).wait()

        def step(j, k, s_dr=True, a_st=True, b_st=True, g_st=True):
            kn1 = (k + 1) % _NB
            kn2 = (k + 2) % _NB
            kn3 = (k + 3) % _NB
            if s_dr:
                s_drain(kn2)
            g_cp(k).wait()
            dst_cp(j, k).wait()
            s_start(k)
            if a_st:
                src_cp(j + 3, kn3).start()
            if b_st:
                dst_cp(j + 1, kn1).start()
            if g_st:
                src_cp(j + 2, kn2).wait()
                g_cp(kn2).start()

        src_cp(0, 0).start()
        src_cp(1, 1).start()
        src_cp(2, 2).start()
        src_cp(0, 0).wait()
        g_cp(0).start()
        src_cp(1, 1).wait()
        g_cp(1).start()
        dst_cp(0, 0).start()
        pltpu.sync_copy(zeros_hbm.at[pl.ds(r0, rpt)], agg_sh.at[pl.ds(r0, rpt)])
        plsc.subcore_barrier()

        step(0, 0, s_dr=False)
        step(1, 1, s_dr=False)

        def body(i, carry):
            j0 = 2 + i * _NB
            for m in range(_NB):
                step(j0 + m, (2 + m) % _NB)
            return carry

        lax.fori_loop(0, (nchunk - 5) // _NB, body, 0)
        j = nchunk - 3
        step(j, j % _NB, a_st=False)
        step(j + 1, (j + 1) % _NB, a_st=False, g_st=False)
        step(j + 2, (j + 2) % _NB, a_st=False, b_st=False, g_st=False)
        s_drain((j + 1) % _NB)
        s_drain((j + 2) % _NB)

        plsc.subcore_barrier()
        pltpu.sync_copy(agg_sh.at[pl.ds(r0, rpt)],
                        out_hbm.at[c, pl.ds(r0, rpt)])

    return segsum


@functools.lru_cache(maxsize=None)
def _make_dense(n, d, h, bn, res):
    def body(*refs):
        it = iter(refs)
        x_ref, agg_ref, w1, b1, w2, b2 = (next(it) for _ in range(6))
        g, b = (next(it), next(it)) if bn else (None, None)
        r_ref = next(it) if res else None
        o_ref = next(it)
        hs = x_ref[...] + agg_ref[0, :n, :] + agg_ref[1, :n, :]
        t = jnp.maximum(
            jnp.dot(hs, w1[...], preferred_element_type=jnp.float32) + b1[...], 0.0)
        y = jnp.dot(t, w2[...], preferred_element_type=jnp.float32) + b2[...]
        if bn:
            m = jnp.mean(y, axis=0, keepdims=True)
            v = jnp.mean((y - m) ** 2, axis=0, keepdims=True)
            y = (y - m) * lax.rsqrt(v + 1e-5) * g[...] + b[...]
            y = jnp.maximum(y, 0.0)
        if res:
            y = y + r_ref[...]
        o_ref[...] = y

    return pl.pallas_call(
        body,
        out_shape=jax.ShapeDtypeStruct((n, d), jnp.float32),
    )


def _conv(x, agg, w1, b1, w2, b2, gamma=None, beta=None, res=None):
    n, d = x.shape
    h = w1.shape[1]
    bn = gamma is not None
    args = [x, agg, w1, b1.reshape(1, h), w2, b2.reshape(1, -1)]
    if bn:
        args += [gamma.reshape(1, -1), beta.reshape(1, -1)]
    if res is not None:
        args += [res]
    return _make_dense(n, d, h, bn, res is not None)(*args)


def kernel(x, edge_index, params):
    n, d = x.shape
    e = edge_index.shape[1]
    src = edge_index[0]
    dst = edge_index[1]
    npad = -(-n // (_NS * 8)) * (_NS * 8)
    zeros = jnp.zeros((npad, d), jnp.float32)
    segsum = _make_segsum(n, e, d)

    def agg_of(xin):
        return segsum(xin, src, dst, zeros)

    p = params["head"]
    out = _conv(x, agg_of(x), p["conv"]["W1"], p["conv"]["b1"],
                p["conv"]["W2"], p["conv"]["b2"], p["gamma"], p["beta"])
    for name in ("res1", "res2", "res3"):
        p = params[name]
        o1 = _conv(out, agg_of(out), p["conv1"]["W1"], p["conv1"]["b1"],
                   p["conv1"]["W2"], p["conv1"]["b2"], p["gamma1"], p["beta1"])
        out = _conv(o1, agg_of(o1), p["conv2"]["W1"], p["conv2"]["b1"],
                    p["conv2"]["W2"], p["conv2"]["b2"], p["gamma2"], p["beta2"],
                    res=out)
    p = params["tail"]["conv"]
    hid = p["W1"].shape[1]
    dout = p["W2"].shape[1]
    w1p = jnp.pad(p["W1"], ((0, 0), (0, d - hid)))
    b1p = jnp.pad(p["b1"], (0, d - hid))
    w2p = jnp.pad(p["W2"], ((0, d - hid), (0, d - dout)))
    b2p = jnp.pad(p["b2"], (0, d - dout))
    y = _conv(out, agg_of(out), w1p, b1p, w2p, b2p)
    return y[:, :dout]
